# baseline (speedup 1.0000x reference)
import functools

import jax
import jax.numpy as jnp
from jax import lax
from jax.experimental import pallas as pl
from jax.experimental.pallas import tpu as pltpu
from jax.experimental.pallas import tpu_sc as plsc

N = 10000
E = 320000
D = 128

NC = 2
NS = 16
NW = NC * NS

EPT = E // NW
ECH = 128
EPT_P = 10240
EIT = EPT_P // ECH
NSUP = 5
SCH = EIT // NSUP

N_PAD = 10240
NPT = N_PAD // NS
ZCH = 128
ZIT = NPT // ZCH

NBUF = 2
TAIL = SCH % NBUF
GB = (SCH - NBUF - TAIL) // NBUF

_MESH = plsc.VectorSubcoreMesh(core_axis_name="c", subcore_axis_name="s",
                               num_cores=NC, num_subcores=NS)


def _zero_fill(buf, rows, width):
    def body(i, carry):
        for j in range(width // 16):
            buf[i, pl.ds(j * 16, 16)] = jnp.zeros((16,), jnp.float32)
        return carry
    lax.fori_loop(0, rows, body, 0)


def _copy_idx(big, i, small):
    for j in range(ECH // 16):
        small[pl.ds(j * 16, 16)] = big[i, pl.ds(j * 16, 16)]


@functools.partial(
    pl.kernel,
    out_type=jax.ShapeDtypeStruct((NC, N_PAD, D), jnp.float32),
    mesh=_MESH,
    scratch_types=[
        pltpu.VMEM((SCH, ECH), jnp.int32),
        pltpu.VMEM((ECH,), jnp.int32),
        pltpu.VMEM((ECH, D), jnp.float32),
        pltpu.VMEM_SHARED((N_PAD, D), jnp.float32),
    ],
)
def _deg_kernel(dst_hbm, out_hbm, idx_v, di_v, ones_v, deg_sh):
    c = lax.axis_index("c")
    s = lax.axis_index("s")
    w = c * NS + s

    _zero_fill(ones_v, ECH, D)
    row0 = s * NPT
    for b in range(ZIT):
        pltpu.sync_copy(ones_v, deg_sh.at[pl.ds(row0 + b * ZCH, ZCH)])

    def fill_ones(i, carry):
        for j in range(D // 16):
            ones_v[i, pl.ds(j * 16, 16)] = jnp.ones((16,), jnp.float32)
        return carry
    lax.fori_loop(0, ECH, fill_ones, 0)
    plsc.subcore_barrier()

    for sup in range(NSUP):
        pltpu.sync_copy(dst_hbm.at[w, sup], idx_v)

        def step(i, carry):
            _copy_idx(idx_v, i, di_v)
            pltpu.sync_copy(ones_v, deg_sh.at[di_v], add=True)
            return carry
        lax.fori_loop(0, SCH, step, 0)
    plsc.subcore_barrier()

    pltpu.sync_copy(deg_sh.at[pl.ds(row0, NPT)],
                    out_hbm.at[c, pl.ds(row0, NPT)])


@functools.partial(
    pl.kernel,
    out_type=jax.ShapeDtypeStruct((NC, N_PAD, D), jnp.float32),
    mesh=_MESH,
    scratch_types=[
        pltpu.VMEM((SCH, ECH), jnp.int32),
        pltpu.VMEM((SCH, ECH), jnp.int32),
        pltpu.VMEM((ECH, D), jnp.float32),
        pltpu.VMEM((ECH, D), jnp.float32),
        pltpu.VMEM((ECH,), jnp.int32),
        pltpu.VMEM((ECH,), jnp.int32),
        pltpu.VMEM((ECH,), jnp.int32),
        pltpu.VMEM_SHARED((N_PAD, D), jnp.float32),
        pltpu.SemaphoreType.DMA,
        pltpu.SemaphoreType.DMA,
    ],
)
def _edge_agg(hs_hbm, src_hbm, dst_hbm, out_hbm,
              idxs_v, idxd_v, r0, r1, si0, si1, di_v, agg_sh,
              g0, g1):
    c = lax.axis_index("c")
    s = lax.axis_index("s")
    w = c * NS + s
    rowbufs = (r0, r1)
    sibufs = (si0, si1)
    gsems = (g0, g1)

    _zero_fill(r0, ECH, D)
    row0 = s * NPT
    for b in range(ZIT):
        pltpu.sync_copy(r0, agg_sh.at[pl.ds(row0 + b * ZCH, ZCH)])
    plsc.subcore_barrier()

    def gather_start(i, b):
        _copy_idx(idxs_v, i, sibufs[b])
        pltpu.async_copy(hs_hbm.at[sibufs[b]], rowbufs[b], gsems[b])

    def gather_wait(b):
        pltpu.make_async_copy(out_hbm.at[0, pl.ds(0, ECH)], rowbufs[b],
                              gsems[b]).wait()

    def scatter(i, b):
        _copy_idx(idxd_v, i, di_v)
        pltpu.sync_copy(rowbufs[b], agg_sh.at[di_v], add=True)

    for sup in range(NSUP):
        pltpu.sync_copy(src_hbm.at[w, sup], idxs_v)
        pltpu.sync_copy(dst_hbm.at[w, sup], idxd_v)

        for b in range(NBUF):
            gather_start(jnp.int32(b), b)

        def body(g, carry):
            i0 = g * NBUF
            for b in range(NBUF):
                i = i0 + b
                gather_wait(b)
                scatter(i, b)
                gather_start(i + NBUF, b)
            return carry
        lax.fori_loop(0, GB, body, 0)

        for b in range(NBUF):
            i = jnp.int32(GB * NBUF + b)
            gather_wait(b)
            scatter(i, b)
        for t in range(TAIL):
            last = jnp.int32(SCH - TAIL + t)
            gather_start(last, t)
            gather_wait(t)
            scatter(last, t)
    plsc.subcore_barrier()

    pltpu.sync_copy(agg_sh.at[pl.ds(row0, NPT)],
                    out_hbm.at[c, pl.ds(row0, NPT)])


RB = 2000
GRID = N // RB


def _prep_body(x_ref, degp_ref, w_ref, hs_ref, dinv_ref):
    deg = degp_ref[0, :, 0:1] + degp_ref[1, :, 0:1] + 1.0
    dinv = lax.rsqrt(deg)
    h = jnp.dot(x_ref[...], w_ref[...], preferred_element_type=jnp.float32)
    hs_ref[...] = h * dinv
    dinv_ref[...] = dinv


def _prep(x, degp, w1):
    return pl.pallas_call(
        _prep_body,
        grid=(GRID,),
        in_specs=[
            pl.BlockSpec((RB, D), lambda i: (i, 0)),
            pl.BlockSpec((NC, RB, D), lambda i: (0, i, 0)),
            pl.BlockSpec((D, D), lambda i: (0, 0)),
        ],
        out_specs=[
            pl.BlockSpec((RB, D), lambda i: (i, 0)),
            pl.BlockSpec((RB, 1), lambda i: (i, 0)),
        ],
        out_shape=[
            jax.ShapeDtypeStruct((N, D), jnp.float32),
            jax.ShapeDtypeStruct((N, 1), jnp.float32),
        ],
    )(x, degp, w1)


def _mid_body(p_ref, hs_ref, dinv_ref, w_ref, b_ref, out_ref):
    dinv = dinv_ref[...]
    agg = p_ref[0] + p_ref[1] + hs_ref[...]
    o1 = dinv * agg + b_ref[...]
    h2 = jnp.dot(o1, w_ref[...], preferred_element_type=jnp.float32)
    out_ref[...] = h2 * dinv


def _mid(parts, hs1, dinv, w2, b1):
    return pl.pallas_call(
        _mid_body,
        grid=(GRID,),
        in_specs=[
            pl.BlockSpec((NC, RB, D), lambda i: (0, i, 0)),
            pl.BlockSpec((RB, D), lambda i: (i, 0)),
            pl.BlockSpec((RB, 1), lambda i: (i, 0)),
            pl.BlockSpec((D, D), lambda i: (0, 0)),
            pl.BlockSpec((1, D), lambda i: (0, 0)),
        ],
        out_specs=pl.BlockSpec((RB, D), lambda i: (i, 0)),
        out_shape=jax.ShapeDtypeStruct((N, D), jnp.float32),
    )(parts, hs1, dinv, w2, b1)


def _fin_body(p_ref, hs_ref, dinv_ref, b_ref, out_ref):
    agg = p_ref[0] + p_ref[1] + hs_ref[...]
    out_ref[...] = dinv_ref[...] * agg + b_ref[...]


def _fin(parts, hs2, dinv, b2):
    return pl.pallas_call(
        _fin_body,
        grid=(GRID,),
        in_specs=[
            pl.BlockSpec((NC, RB, D), lambda i: (0, i, 0)),
            pl.BlockSpec((RB, D), lambda i: (i, 0)),
            pl.BlockSpec((RB, 1), lambda i: (i, 0)),
            pl.BlockSpec((1, D), lambda i: (0, 0)),
        ],
        out_specs=pl.BlockSpec((RB, D), lambda i: (i, 0)),
        out_shape=jax.ShapeDtypeStruct((N, D), jnp.float32),
    )(parts, hs2, dinv, b2)


def kernel(x, edge_index, W1, b1, W2, b2):
    src2 = edge_index[0].astype(jnp.int32).reshape(NW, EPT)
    dst2 = edge_index[1].astype(jnp.int32).reshape(NW, EPT)
    pad = EPT_P - EPT
    src4 = jnp.pad(src2, ((0, 0), (0, pad))).reshape(NW, NSUP, SCH, ECH)
    dst4 = jnp.pad(dst2, ((0, 0), (0, pad)),
                   constant_values=N_PAD - 1).reshape(NW, NSUP, SCH, ECH)

    degp = _deg_kernel(dst4)
    hs1, dinv = _prep(x, degp, W1)
    parts1 = _edge_agg(hs1, src4, dst4)
    hs2 = _mid(parts1, hs1, dinv, W2, b1.reshape(1, D))
    parts2 = _edge_agg(hs2, src4, dst4)
    return _fin(parts2, hs2, dinv, b2.reshape(1, D))


# speedup vs baseline: 2.3722x; 2.3722x over previous
import functools

import jax
import jax.numpy as jnp
from jax import lax
from jax.experimental import pallas as pl
from jax.experimental.pallas import tpu as pltpu
from jax.experimental.pallas import tpu_sc as plsc

N = 10000
E = 320000
D = 128

NC = 2
NS = 16
NW = NC * NS

EPT = E // NW
ECH = 80
EIT = EPT // ECH
NSUP = 5
SCH = EIT // NSUP

N_PAD = 10240
NPT = N_PAD // NS
ZCH = 32
ZIT = NPT // ZCH

NBUF = 2
GB = (SCH - NBUF - 1) // NBUF

_MESH = plsc.VectorSubcoreMesh(core_axis_name="c", subcore_axis_name="s",
                               num_cores=NC, num_subcores=NS)


def _zero_fill(buf, rows, width):
    def body(i, carry):
        for j in range(width // 16):
            buf[i, pl.ds(j * 16, 16)] = jnp.zeros((16,), jnp.float32)
        return carry
    lax.fori_loop(0, rows, body, 0)


def _copy_idx(big, i, small):
    for j in range(ECH // 16):
        small[pl.ds(j * 16, 16)] = big[i, pl.ds(j * 16, 16)]


@functools.partial(
    pl.kernel,
    out_type=jax.ShapeDtypeStruct((NC, N_PAD, D), jnp.float32),
    mesh=_MESH,
    scratch_types=[
        pltpu.VMEM((SCH, ECH), jnp.int32),
        pltpu.VMEM((ECH,), jnp.int32),
        pltpu.VMEM((ECH, D), jnp.float32),
        pltpu.VMEM((ZCH, D), jnp.float32),
        pltpu.VMEM_SHARED((N_PAD, D), jnp.float32),
    ],
)
def _deg_kernel(dst_hbm, out_hbm, idx_v, di_v, ones_v, stage_v, deg_sh):
    c = lax.axis_index("c")
    s = lax.axis_index("s")
    w = c * NS + s

    def fill_ones(i, carry):
        for j in range(D // 16):
            ones_v[i, pl.ds(j * 16, 16)] = jnp.ones((16,), jnp.float32)
        return carry
    lax.fori_loop(0, ECH, fill_ones, 0)
    _zero_fill(stage_v, ZCH, D)

    row0 = s * NPT
    for b in range(ZIT):
        pltpu.sync_copy(stage_v, deg_sh.at[pl.ds(row0 + b * ZCH, ZCH)])
    plsc.subcore_barrier()

    for sup in range(NSUP):
        pltpu.sync_copy(dst_hbm.at[w, sup], idx_v)

        def step(i, carry):
            _copy_idx(idx_v, i, di_v)
            pltpu.sync_copy(ones_v, deg_sh.at[di_v], add=True)
            return carry
        lax.fori_loop(0, SCH, step, 0)
    plsc.subcore_barrier()

    pltpu.sync_copy(deg_sh.at[pl.ds(row0, NPT)],
                    out_hbm.at[c, pl.ds(row0, NPT)])


@functools.partial(
    pl.kernel,
    out_type=jax.ShapeDtypeStruct((NC, N_PAD, D), jnp.float32),
    mesh=_MESH,
    scratch_types=[
        pltpu.VMEM((SCH, ECH), jnp.int32),
        pltpu.VMEM((SCH, ECH), jnp.int32),
        pltpu.VMEM((ECH, D), jnp.float32),
        pltpu.VMEM((ECH, D), jnp.float32),
        pltpu.VMEM((ECH,), jnp.int32),
        pltpu.VMEM((ECH,), jnp.int32),
        pltpu.VMEM((ECH,), jnp.int32),
        pltpu.VMEM((ZCH, D), jnp.float32),
        pltpu.VMEM_SHARED((N_PAD, D), jnp.float32),
        pltpu.SemaphoreType.DMA,
        pltpu.SemaphoreType.DMA,
    ],
)
def _edge_agg(hs_hbm, src_hbm, dst_hbm, out_hbm,
              idxs_v, idxd_v, r0, r1, si0, si1, di_v, stage_v, agg_sh,
              g0, g1):
    c = lax.axis_index("c")
    s = lax.axis_index("s")
    w = c * NS + s
    rowbufs = (r0, r1)
    sibufs = (si0, si1)
    gsems = (g0, g1)

    _zero_fill(stage_v, ZCH, D)
    row0 = s * NPT
    for b in range(ZIT):
        pltpu.sync_copy(stage_v, agg_sh.at[pl.ds(row0 + b * ZCH, ZCH)])
    plsc.subcore_barrier()

    def gather_start(i, b):
        _copy_idx(idxs_v, i, sibufs[b])
        pltpu.async_copy(hs_hbm.at[sibufs[b]], rowbufs[b], gsems[b])

    def gather_wait(b):
        pltpu.make_async_copy(out_hbm.at[0, pl.ds(0, ECH)], rowbufs[b],
                              gsems[b]).wait()

    def scatter(i, b):
        _copy_idx(idxd_v, i, di_v)
        pltpu.sync_copy(rowbufs[b], agg_sh.at[di_v], add=True)

    for sup in range(NSUP):
        pltpu.sync_copy(src_hbm.at[w, sup], idxs_v)
        pltpu.sync_copy(dst_hbm.at[w, sup], idxd_v)

        for b in range(NBUF):
            gather_start(jnp.int32(b), b)

        def body(g, carry):
            i0 = g * NBUF
            for b in range(NBUF):
                i = i0 + b
                gather_wait(b)
                scatter(i, b)
                gather_start(i + NBUF, b)
            return carry
        lax.fori_loop(0, GB, body, 0)

        for b in range(NBUF):
            i = jnp.int32(GB * NBUF + b)
            gather_wait(b)
            scatter(i, b)
        last = jnp.int32(SCH - 1)
        gather_start(last, 0)
        gather_wait(0)
        scatter(last, 0)
    plsc.subcore_barrier()

    pltpu.sync_copy(agg_sh.at[pl.ds(row0, NPT)],
                    out_hbm.at[c, pl.ds(row0, NPT)])


RB = 2000
GRID = N // RB


def _prep_body(x_ref, degp_ref, w_ref, hs_ref, dinv_ref):
    deg = degp_ref[0, :, 0:1] + degp_ref[1, :, 0:1] + 1.0
    dinv = lax.rsqrt(deg)
    h = jnp.dot(x_ref[...], w_ref[...], preferred_element_type=jnp.float32)
    hs_ref[...] = h * dinv
    dinv_ref[...] = dinv


def _prep(x, degp, w1):
    return pl.pallas_call(
        _prep_body,
        grid=(GRID,),
        in_specs=[
            pl.BlockSpec((RB, D), lambda i: (i, 0)),
            pl.BlockSpec((NC, RB, D), lambda i: (0, i, 0)),
            pl.BlockSpec((D, D), lambda i: (0, 0)),
        ],
        out_specs=[
            pl.BlockSpec((RB, D), lambda i: (i, 0)),
            pl.BlockSpec((RB, 1), lambda i: (i, 0)),
        ],
        out_shape=[
            jax.ShapeDtypeStruct((N, D), jnp.float32),
            jax.ShapeDtypeStruct((N, 1), jnp.float32),
        ],
    )(x, degp, w1)


def _mid_body(p_ref, hs_ref, dinv_ref, w_ref, b_ref, out_ref):
    dinv = dinv_ref[...]
    agg = p_ref[0] + p_ref[1] + hs_ref[...]
    o1 = dinv * agg + b_ref[...]
    h2 = jnp.dot(o1, w_ref[...], preferred_element_type=jnp.float32)
    out_ref[...] = h2 * dinv


def _mid(parts, hs1, dinv, w2, b1):
    return pl.pallas_call(
        _mid_body,
        grid=(GRID,),
        in_specs=[
            pl.BlockSpec((NC, RB, D), lambda i: (0, i, 0)),
            pl.BlockSpec((RB, D), lambda i: (i, 0)),
            pl.BlockSpec((RB, 1), lambda i: (i, 0)),
            pl.BlockSpec((D, D), lambda i: (0, 0)),
            pl.BlockSpec((1, D), lambda i: (0, 0)),
        ],
        out_specs=pl.BlockSpec((RB, D), lambda i: (i, 0)),
        out_shape=jax.ShapeDtypeStruct((N, D), jnp.float32),
    )(parts, hs1, dinv, w2, b1)


def _fin_body(p_ref, hs_ref, dinv_ref, b_ref, out_ref):
    agg = p_ref[0] + p_ref[1] + hs_ref[...]
    out_ref[...] = dinv_ref[...] * agg + b_ref[...]


def _fin(parts, hs2, dinv, b2):
    return pl.pallas_call(
        _fin_body,
        grid=(GRID,),
        in_specs=[
            pl.BlockSpec((NC, RB, D), lambda i: (0, i, 0)),
            pl.BlockSpec((RB, D), lambda i: (i, 0)),
            pl.BlockSpec((RB, 1), lambda i: (i, 0)),
            pl.BlockSpec((1, D), lambda i: (0, 0)),
        ],
        out_specs=pl.BlockSpec((RB, D), lambda i: (i, 0)),
        out_shape=jax.ShapeDtypeStruct((N, D), jnp.float32),
    )(parts, hs2, dinv, b2)


def kernel(x, edge_index, W1, b1, W2, b2):
    src4 = edge_index[0].astype(jnp.int32).reshape(NW, NSUP, SCH, ECH)
    dst4 = edge_index[1].astype(jnp.int32).reshape(NW, NSUP, SCH, ECH)

    degp = _deg_kernel(dst4)
    hs1, dinv = _prep(x, degp, W1)
    parts1 = _edge_agg(hs1, src4, dst4)
    hs2 = _mid(parts1, hs1, dinv, W2, b1.reshape(1, D))
    parts2 = _edge_agg(hs2, src4, dst4)
    return _fin(parts2, hs2, dinv, b2.reshape(1, D))


# speedup vs baseline: 2.6274x; 1.1076x over previous
import functools

import jax
import jax.numpy as jnp
from jax import lax
from jax.experimental import pallas as pl
from jax.experimental.pallas import tpu as pltpu
from jax.experimental.pallas import tpu_sc as plsc

N = 10000
E = 320000
D = 128

NC = 2
NS = 16
NW = NC * NS

EPT = E // NW
ECH = 80
EIT = EPT // ECH
NSUP = 5
SCH = EIT // NSUP

N_PAD = 10240
NPT = N_PAD // NS
ZCH = 32
ZIT = NPT // ZCH

NBUF = 3
TAIL = SCH % NBUF
GB = (SCH - NBUF - TAIL) // NBUF

_MESH = plsc.VectorSubcoreMesh(core_axis_name="c", subcore_axis_name="s",
                               num_cores=NC, num_subcores=NS)


def _zero_fill(buf, rows, width):
    def body(i, carry):
        for j in range(width // 16):
            buf[i, pl.ds(j * 16, 16)] = jnp.zeros((16,), jnp.float32)
        return carry
    lax.fori_loop(0, rows, body, 0)


def _copy_idx(big, i, small):
    for j in range(ECH // 16):
        small[pl.ds(j * 16, 16)] = big[i, pl.ds(j * 16, 16)]


@functools.partial(
    pl.kernel,
    out_type=jax.ShapeDtypeStruct((NC, N_PAD, D), jnp.float32),
    mesh=_MESH,
    scratch_types=[
        pltpu.VMEM((SCH, ECH), jnp.int32),
        pltpu.VMEM((ECH,), jnp.int32),
        pltpu.VMEM((ECH, D), jnp.float32),
        pltpu.VMEM((ZCH, D), jnp.float32),
        pltpu.VMEM_SHARED((N_PAD, D), jnp.float32),
    ],
)
def _deg_kernel(dst_hbm, out_hbm, idx_v, di_v, ones_v, stage_v, deg_sh):
    c = lax.axis_index("c")
    s = lax.axis_index("s")
    w = c * NS + s

    def fill_ones(i, carry):
        for j in range(D // 16):
            ones_v[i, pl.ds(j * 16, 16)] = jnp.ones((16,), jnp.float32)
        return carry
    lax.fori_loop(0, ECH, fill_ones, 0)
    _zero_fill(stage_v, ZCH, D)

    row0 = s * NPT
    for b in range(ZIT):
        pltpu.sync_copy(stage_v, deg_sh.at[pl.ds(row0 + b * ZCH, ZCH)])
    plsc.subcore_barrier()

    for sup in range(NSUP):
        pltpu.sync_copy(dst_hbm.at[w, sup], idx_v)

        def step(i, carry):
            _copy_idx(idx_v, i, di_v)
            pltpu.sync_copy(ones_v, deg_sh.at[di_v], add=True)
            return carry
        lax.fori_loop(0, SCH, step, 0)
    plsc.subcore_barrier()

    pltpu.sync_copy(deg_sh.at[pl.ds(row0, NPT)],
                    out_hbm.at[c, pl.ds(row0, NPT)])


@functools.partial(
    pl.kernel,
    out_type=jax.ShapeDtypeStruct((NC, N_PAD, D), jnp.float32),
    mesh=_MESH,
    scratch_types=[
        pltpu.VMEM((SCH, ECH), jnp.int32),
        pltpu.VMEM((SCH, ECH), jnp.int32),
        pltpu.VMEM((ECH, D), jnp.float32),
        pltpu.VMEM((ECH, D), jnp.float32),
        pltpu.VMEM((ECH, D), jnp.float32),
        pltpu.VMEM((ECH,), jnp.int32),
        pltpu.VMEM((ECH,), jnp.int32),
        pltpu.VMEM((ECH,), jnp.int32),
        pltpu.VMEM((ECH,), jnp.int32),
        pltpu.VMEM_SHARED((N_PAD, D), jnp.float32),
        pltpu.SemaphoreType.DMA,
        pltpu.SemaphoreType.DMA,
        pltpu.SemaphoreType.DMA,
    ],
)
def _edge_agg(hs_hbm, src_hbm, dst_hbm, out_hbm,
              idxs_v, idxd_v, r0, r1, r2, si0, si1, si2, di_v, agg_sh,
              g0, g1, g2):
    c = lax.axis_index("c")
    s = lax.axis_index("s")
    w = c * NS + s
    rowbufs = (r0, r1, r2)
    sibufs = (si0, si1, si2)
    gsems = (g0, g1, g2)

    _zero_fill(r0, ECH, D)
    row0 = s * NPT
    for b in range(NPT // ECH):
        pltpu.sync_copy(r0, agg_sh.at[pl.ds(row0 + b * ECH, ECH)])
    plsc.subcore_barrier()

    def gather_start(i, b):
        _copy_idx(idxs_v, i, sibufs[b])
        pltpu.async_copy(hs_hbm.at[sibufs[b]], rowbufs[b], gsems[b])

    def gather_wait(b):
        pltpu.make_async_copy(out_hbm.at[0, pl.ds(0, ECH)], rowbufs[b],
                              gsems[b]).wait()

    def scatter(i, b):
        _copy_idx(idxd_v, i, di_v)
        pltpu.sync_copy(rowbufs[b], agg_sh.at[di_v], add=True)

    for sup in range(NSUP):
        pltpu.sync_copy(src_hbm.at[w, sup], idxs_v)
        pltpu.sync_copy(dst_hbm.at[w, sup], idxd_v)

        for b in range(NBUF):
            gather_start(jnp.int32(b), b)

        def body(g, carry):
            i0 = g * NBUF
            for b in range(NBUF):
                i = i0 + b
                gather_wait(b)
                scatter(i, b)
                gather_start(i + NBUF, b)
            return carry
        lax.fori_loop(0, GB, body, 0)

        for b in range(NBUF):
            i = jnp.int32(GB * NBUF + b)
            gather_wait(b)
            scatter(i, b)
        for t in range(TAIL):
            last = jnp.int32(SCH - TAIL + t)
            gather_start(last, t)
            gather_wait(t)
            scatter(last, t)
    plsc.subcore_barrier()

    pltpu.sync_copy(agg_sh.at[pl.ds(row0, NPT)],
                    out_hbm.at[c, pl.ds(row0, NPT)])


RB = 2000
GRID = N // RB


def _prep_body(x_ref, degp_ref, w_ref, hs_ref, dinv_ref):
    deg = degp_ref[0, :, 0:1] + degp_ref[1, :, 0:1] + 1.0
    dinv = lax.rsqrt(deg)
    h = jnp.dot(x_ref[...], w_ref[...], preferred_element_type=jnp.float32)
    hs_ref[...] = h * dinv
    dinv_ref[...] = dinv


def _prep(x, degp, w1):
    return pl.pallas_call(
        _prep_body,
        grid=(GRID,),
        in_specs=[
            pl.BlockSpec((RB, D), lambda i: (i, 0)),
            pl.BlockSpec((NC, RB, D), lambda i: (0, i, 0)),
            pl.BlockSpec((D, D), lambda i: (0, 0)),
        ],
        out_specs=[
            pl.BlockSpec((RB, D), lambda i: (i, 0)),
            pl.BlockSpec((RB, 1), lambda i: (i, 0)),
        ],
        out_shape=[
            jax.ShapeDtypeStruct((N, D), jnp.float32),
            jax.ShapeDtypeStruct((N, 1), jnp.float32),
        ],
    )(x, degp, w1)


def _mid_body(p_ref, hs_ref, dinv_ref, w_ref, b_ref, out_ref):
    dinv = dinv_ref[...]
    agg = p_ref[0] + p_ref[1] + hs_ref[...]
    o1 = dinv * agg + b_ref[...]
    h2 = jnp.dot(o1, w_ref[...], preferred_element_type=jnp.float32)
    out_ref[...] = h2 * dinv


def _mid(parts, hs1, dinv, w2, b1):
    return pl.pallas_call(
        _mid_body,
        grid=(GRID,),
        in_specs=[
            pl.BlockSpec((NC, RB, D), lambda i: (0, i, 0)),
            pl.BlockSpec((RB, D), lambda i: (i, 0)),
            pl.BlockSpec((RB, 1), lambda i: (i, 0)),
            pl.BlockSpec((D, D), lambda i: (0, 0)),
            pl.BlockSpec((1, D), lambda i: (0, 0)),
        ],
        out_specs=pl.BlockSpec((RB, D), lambda i: (i, 0)),
        out_shape=jax.ShapeDtypeStruct((N, D), jnp.float32),
    )(parts, hs1, dinv, w2, b1)


def _fin_body(p_ref, hs_ref, dinv_ref, b_ref, out_ref):
    agg = p_ref[0] + p_ref[1] + hs_ref[...]
    out_ref[...] = dinv_ref[...] * agg + b_ref[...]


def _fin(parts, hs2, dinv, b2):
    return pl.pallas_call(
        _fin_body,
        grid=(GRID,),
        in_specs=[
            pl.BlockSpec((NC, RB, D), lambda i: (0, i, 0)),
            pl.BlockSpec((RB, D), lambda i: (i, 0)),
            pl.BlockSpec((RB, 1), lambda i: (i, 0)),
            pl.BlockSpec((1, D), lambda i: (0, 0)),
        ],
        out_specs=pl.BlockSpec((RB, D), lambda i: (i, 0)),
        out_shape=jax.ShapeDtypeStruct((N, D), jnp.float32),
    )(parts, hs2, dinv, b2)


def kernel(x, edge_index, W1, b1, W2, b2):
    src4 = edge_index[0].astype(jnp.int32).reshape(NW, NSUP, SCH, ECH)
    dst4 = edge_index[1].astype(jnp.int32).reshape(NW, NSUP, SCH, ECH)

    degp = _deg_kernel(dst4)
    hs1, dinv = _prep(x, degp, W1)
    parts1 = _edge_agg(hs1, src4, dst4)
    hs2 = _mid(parts1, hs1, dinv, W2, b1.reshape(1, D))
    parts2 = _edge_agg(hs2, src4, dst4)
    return _fin(parts2, hs2, dinv, b2.reshape(1, D))
